# baseline (speedup 1.0000x reference)
import jax
import jax.numpy as jnp
from jax.experimental import pallas as pl
from jax.experimental.pallas import tpu as pltpu
from jax.experimental.pallas import tpu_sc as plsc

B, S, D, C = 1024, 200, 128, 49
SS = 8
NS = S // SS
_GATHER_WINDOW = 128


def _sc_gather(x2d, indices):
    n_idx = indices.shape[0]
    indices = indices.reshape(1, n_idx)
    mesh = plsc.VectorSubcoreMesh(core_axis_name="core",
                                  subcore_axis_name="subcore")

    @pl.kernel(out_type=jax.ShapeDtypeStruct((n_idx, D), x2d.dtype),
               mesh=mesh)
    def gather_kernel(x_hbm, i_hbm, o_hbm):
        def body(i_vmem, o_vmem):
            pltpu.sync_copy(x_hbm.at[i_vmem.at[0]], o_vmem)

        pltpu.emit_pipeline(
            body,
            grid=(n_idx // _GATHER_WINDOW,),
            in_specs=[pl.BlockSpec((1, _GATHER_WINDOW),
                                   index_map=lambda i: (0, i))],
            out_specs=[pl.BlockSpec((_GATHER_WINDOW, D),
                                    index_map=lambda i: (i, 0))],
            core_axis_name="subcore",
            dimension_semantics=(pltpu.PARALLEL,),
        )(i_hbm, o_hbm)

    return gather_kernel(x2d, indices)


def _body(vs_ref, ve_ref, x_ref, w1_ref, b1_ref, w2_ref, b2_ref,
          ps_ref, pe_ref, out1_ref, out2_ref, d1_ref, d2_ref):
    i = pl.program_id(0)
    dn = (((0,), (1,)), ((), ()))

    @pl.when(i == 0)
    def _():
        v = 0.5 * (vs_ref[...] + ve_ref[...])
        d1_ref[...] = jax.lax.dot_general(
            w1_ref[...], v, dn, preferred_element_type=jnp.float32)
        d2_ref[...] = jax.lax.dot_general(
            w2_ref[...], v, dn, preferred_element_type=jnp.float32)

    pos_s = ps_ref[0, 0, :].reshape(1, B)
    pos_e = pe_ref[0, 0, :].reshape(1, B)
    iota_s = i * SS + jax.lax.broadcasted_iota(jnp.int32, (SS, B), 0)
    coef = ((iota_s == pos_s) | (iota_s == pos_e)).astype(jnp.float32)

    xt = jnp.swapaxes(x_ref[...], 0, 1).reshape(SS * B, D)
    for w_ref, b_ref, d_ref, out_ref in (
            (w1_ref, b1_ref, d1_ref, out1_ref),
            (w2_ref, b2_ref, d2_ref, out2_ref)):
        p = jax.lax.dot_general(w_ref[...], xt, dn,
                                preferred_element_type=jnp.float32)
        p3 = p.reshape(C, SS, B) + b_ref[...].reshape(C, 1, 1)
        p3 = p3 + d_ref[...].reshape(C, 1, B) * coef.reshape(1, SS, B)
        out_ref[...] = 0.5 * jnp.tanh(0.5 * p3) + 0.5


def _kernel_impl(x_lstm, position_s, position_e, W1, b1, W2, b2):
    b1r = b1.reshape(C, 1)
    b2r = b2.reshape(C, 1)
    pos_s = position_s.astype(jnp.int32)
    pos_e = position_e.astype(jnp.int32)

    x2d = x_lstm.reshape(B * S, D)
    row_ids = jnp.arange(B, dtype=jnp.int32) * S
    gathered = _sc_gather(x2d, jnp.concatenate([row_ids + pos_s,
                                                row_ids + pos_e]))

    pos_s3 = pos_s.reshape(1, 1, B)
    pos_e3 = pos_e.reshape(1, 1, B)
    out1, out2 = pl.pallas_call(
        _body,
        grid=(NS,),
        in_specs=[
            pl.BlockSpec((B, D), lambda i: (0, 0)),
            pl.BlockSpec((B, D), lambda i: (1, 0)),
            pl.BlockSpec((B, SS, D), lambda i: (0, i, 0)),
            pl.BlockSpec((D, C), lambda i: (0, 0)),
            pl.BlockSpec((C, 1), lambda i: (0, 0)),
            pl.BlockSpec((D, C), lambda i: (0, 0)),
            pl.BlockSpec((C, 1), lambda i: (0, 0)),
            pl.BlockSpec((1, 1, B), lambda i: (0, 0, 0)),
            pl.BlockSpec((1, 1, B), lambda i: (0, 0, 0)),
        ],
        out_specs=[
            pl.BlockSpec((C, SS, B), lambda i: (0, i, 0)),
            pl.BlockSpec((C, SS, B), lambda i: (0, i, 0)),
        ],
        scratch_shapes=[
            pltpu.VMEM((C, B), jnp.float32),
            pltpu.VMEM((C, B), jnp.float32),
        ],
        out_shape=[
            jax.ShapeDtypeStruct((C, S, B), jnp.float32),
            jax.ShapeDtypeStruct((C, S, B), jnp.float32),
        ],
        compiler_params=pltpu.CompilerParams(
            dimension_semantics=("arbitrary",),
        ),
    )(gathered, gathered, x_lstm, W1, b1r, W2, b2r, pos_s3, pos_e3)
    return (jnp.transpose(out1, (2, 1, 0)), jnp.transpose(out2, (2, 1, 0)))


kernel = jax.jit(_kernel_impl)


# speedup vs baseline: 1.2813x; 1.2813x over previous
import jax
import jax.numpy as jnp
from jax.experimental import pallas as pl
from jax.experimental.pallas import tpu as pltpu
from jax.experimental.pallas import tpu_sc as plsc

B, S, D, C = 1024, 200, 128, 49
BB = 256
G = B // BB
NS = 5
SS = S // NS
_GATHER_WINDOW = 128


def _sc_gather(x2d, indices):
    n_idx = indices.shape[0]
    indices = indices.reshape(1, n_idx)
    mesh = plsc.VectorSubcoreMesh(core_axis_name="core",
                                  subcore_axis_name="subcore")

    @pl.kernel(out_type=jax.ShapeDtypeStruct((n_idx, D), x2d.dtype),
               mesh=mesh)
    def gather_kernel(x_hbm, i_hbm, o_hbm):
        def body(i_vmem, o_vmem):
            pltpu.sync_copy(x_hbm.at[i_vmem.at[0]], o_vmem)

        pltpu.emit_pipeline(
            body,
            grid=(n_idx // _GATHER_WINDOW,),
            in_specs=[pl.BlockSpec((1, _GATHER_WINDOW),
                                   index_map=lambda i: (0, i))],
            out_specs=[pl.BlockSpec((_GATHER_WINDOW, D),
                                    index_map=lambda i: (i, 0))],
            core_axis_name="subcore",
            dimension_semantics=(pltpu.PARALLEL,),
        )(i_hbm, o_hbm)

    return gather_kernel(x2d, indices)


def _body(vs_ref, ve_ref, x_ref, w1_ref, b1_ref, w2_ref, b2_ref,
          ps_ref, pe_ref, out1_ref, out2_ref):
    sb = pl.program_id(1)
    dn = (((0,), (1,)), ((), ()))
    v = 0.5 * (vs_ref[...] + ve_ref[...])

    pos_s = ps_ref[0, 0, :].reshape(1, BB)
    pos_e = pe_ref[0, 0, :].reshape(1, BB)
    iota_s = sb * SS + jax.lax.broadcasted_iota(jnp.int32, (SS, BB), 0)
    coef = ((iota_s == pos_s) | (iota_s == pos_e)).astype(jnp.float32)

    xt = jnp.swapaxes(x_ref[...], 0, 1).reshape(SS * BB, D)
    for w_ref, b_ref, out_ref in ((w1_ref, b1_ref, out1_ref),
                                  (w2_ref, b2_ref, out2_ref)):
        p = jax.lax.dot_general(w_ref[...], xt, dn,
                                preferred_element_type=jnp.float32)
        delta = jax.lax.dot_general(w_ref[...], v, dn,
                                    preferred_element_type=jnp.float32)
        p3 = p.reshape(C, SS, BB) + b_ref[...].reshape(C, 1, 1)
        p3 = p3 + delta.reshape(C, 1, BB) * coef.reshape(1, SS, BB)
        out_ref[...] = 0.5 * jnp.tanh(0.5 * p3) + 0.5


def _kernel_impl(x_lstm, position_s, position_e, W1, b1, W2, b2):
    b1r = b1.reshape(C, 1)
    b2r = b2.reshape(C, 1)
    pos_s = position_s.astype(jnp.int32)
    pos_e = position_e.astype(jnp.int32)

    x2d = x_lstm.reshape(B * S, D)
    row_ids = jnp.arange(B, dtype=jnp.int32) * S
    gathered = _sc_gather(x2d, jnp.concatenate([row_ids + pos_s,
                                                row_ids + pos_e]))

    pos_s3 = pos_s.reshape(G, 1, BB)
    pos_e3 = pos_e.reshape(G, 1, BB)
    out1, out2 = pl.pallas_call(
        _body,
        grid=(G, NS),
        in_specs=[
            pl.BlockSpec((BB, D), lambda g, sb: (g, 0)),
            pl.BlockSpec((BB, D), lambda g, sb: (G + g, 0)),
            pl.BlockSpec((BB, SS, D), lambda g, sb: (g, sb, 0)),
            pl.BlockSpec((D, C), lambda g, sb: (0, 0)),
            pl.BlockSpec((C, 1), lambda g, sb: (0, 0)),
            pl.BlockSpec((D, C), lambda g, sb: (0, 0)),
            pl.BlockSpec((C, 1), lambda g, sb: (0, 0)),
            pl.BlockSpec((1, 1, BB), lambda g, sb: (g, 0, 0)),
            pl.BlockSpec((1, 1, BB), lambda g, sb: (g, 0, 0)),
        ],
        out_specs=[
            pl.BlockSpec((C, SS, BB), lambda g, sb: (0, sb, g)),
            pl.BlockSpec((C, SS, BB), lambda g, sb: (0, sb, g)),
        ],
        out_shape=[
            jax.ShapeDtypeStruct((C, S, B), jnp.float32),
            jax.ShapeDtypeStruct((C, S, B), jnp.float32),
        ],
        compiler_params=pltpu.CompilerParams(
            dimension_semantics=("parallel", "parallel"),
        ),
    )(gathered, gathered, x_lstm, W1, b1r, W2, b2r, pos_s3, pos_e3)
    return (jnp.transpose(out1, (2, 1, 0)), jnp.transpose(out2, (2, 1, 0)))


kernel = jax.jit(_kernel_impl)
